# baseline (speedup 1.0000x reference)
import functools
import math

import jax
import jax.numpy as jnp
from jax import lax
from jax.experimental import pallas as pl
from jax.experimental.pallas import tpu as pltpu
from jax.experimental.pallas import tpu_sc as plsc

N = 50000
E = 800000
IN_C = 128
HID = 64
OUT_C = 64
NUM_LAYERS = 4
N_NETS = 3
ALPHA = 0.1
THETA = 0.5

HALF = HID // 2
NSUB = 16
BLK = 112
K = 3
NCHUNK = 149
NBLK = K * NCHUNK
E_PAD = NSUB * NBLK * BLK
SRC_EXTRA = 2 * K * BLK
ROWS_PER_TILE = 3126
R_PAD = NSUB * ROWS_PER_TILE
DUMMY = 50008
RBLK = 50
RT = N // RBLK

_MESH = plsc.VectorSubcoreMesh(
    core_axis_name="c", subcore_axis_name="s", num_cores=2, num_subcores=16
)


def _sc_body(xl_hbm, xr_hbm, cidx_hbm, zeros_hbm, outl_hbm, outr_hbm,
             cidx, rows_bf, rows, acc, gsem, ssem, isem):
    c = lax.axis_index("c")
    s = lax.axis_index("s")

    r0 = s * ROWS_PER_TILE
    pltpu.sync_copy(zeros_hbm, acc.at[pl.ds(r0, ROWS_PER_TILE)])

    plsc.subcore_barrier()

    def edge_loop(x_hbm):
        rbase = s * NBLK

        def load_cidx(t, slot):
            return pltpu.async_copy(
                cidx_hbm.at[pl.ds(rbase + t * K, K)], cidx.at[slot], isem)

        def drain_cidx():
            pltpu.make_async_copy(
                cidx_hbm.at[pl.ds(0, K)], cidx.at[0], isem).wait()

        def fire_gathers(islot):
            for j in range(K):
                pltpu.async_copy(x_hbm.at[cidx.at[islot, j, 0]],
                                 rows_bf.at[j], gsem)

        def drain_gathers():
            for j in range(K):
                pltpu.make_async_copy(x_hbm.at[pl.ds(0, BLK)],
                                      rows_bf.at[j], gsem).wait()

        def convert(slot):
            mask = jnp.int32(-65536)
            for j in range(K):
                @pl.loop(0, BLK, step=8)
                def _cv(r):
                    for u in range(8):
                        w = rows_bf[j, r + u, pl.ds(0, 16)]
                        a = plsc.bitcast(jnp.left_shift(w, 16), jnp.float32)
                        b = plsc.bitcast(jnp.bitwise_and(w, mask), jnp.float32)
                        rows[slot, j, r + u, pl.ds(0, 16)] = a
                        rows[slot, j, r + u, pl.ds(16, 16)] = b

        def fire_scatters(slot, islot):
            for j in range(K):
                pltpu.async_copy(rows.at[slot, j], acc.at[cidx.at[islot, j, 1]],
                                 ssem, add=True)

        def drain_scatters(slot):
            for j in range(K):
                pltpu.make_async_copy(zeros_hbm.at[pl.ds(0, BLK)],
                                      rows.at[slot, j], ssem).wait()

        load_cidx(0, 0).wait()
        fire_gathers(0)
        load_cidx(1, 1)

        @pl.loop(0, NCHUNK - 1)
        def _ch(t):
            q = t % 2
            q3 = t % 3

            drain_gathers()
            convert(q)

            @pl.when(t != 0)
            def _ds():
                drain_scatters(1 - q)

            fire_scatters(q, q3)
            drain_cidx()
            fire_gathers((t + 1) % 3)
            load_cidx(t + 2, (t + 2) % 3)

        L = NCHUNK - 1
        drain_gathers()
        convert(L % 2)
        drain_scatters(1 - L % 2)
        fire_scatters(L % 2, L % 3)
        drain_scatters(L % 2)
        drain_cidx()

    @pl.when(c == 0)
    def _e0():
        edge_loop(xl_hbm)

    @pl.when(c == 1)
    def _e1():
        edge_loop(xr_hbm)

    plsc.subcore_barrier()

    @pl.when(c == 0)
    def _f0():
        pltpu.sync_copy(acc.at[pl.ds(r0, ROWS_PER_TILE)],
                        outl_hbm.at[pl.ds(r0, ROWS_PER_TILE)])

    @pl.when(c == 1)
    def _f1():
        pltpu.sync_copy(acc.at[pl.ds(r0, ROWS_PER_TILE)],
                        outr_hbm.at[pl.ds(r0, ROWS_PER_TILE)])


_sc_propagate = pl.kernel(
    _sc_body,
    out_type=[jax.ShapeDtypeStruct((R_PAD, HALF), jnp.float32),
              jax.ShapeDtypeStruct((R_PAD, HALF), jnp.float32)],
    mesh=_MESH,
    scratch_types=[
        pltpu.VMEM((3, K, 2, BLK), jnp.int32),
        pltpu.VMEM((K, BLK, HALF // 2), jnp.int32),
        pltpu.VMEM((2, K, BLK, HALF), jnp.float32),
        pltpu.VMEM_SHARED((R_PAD, HALF), jnp.float32),
        pltpu.SemaphoreType.DMA,
        pltpu.SemaphoreType.DMA,
        pltpu.SemaphoreType.DMA,
    ],
    compiler_params=pltpu.CompilerParams(use_tc_tiling_on_sc=False,
                                         needs_layout_passes=False),
)


def _pack_i32(a, b):
    ai = jax.lax.bitcast_convert_type(
        a.astype(jnp.bfloat16).astype(jnp.float32), jnp.int32)
    bi = jax.lax.bitcast_convert_type(
        b.astype(jnp.bfloat16).astype(jnp.float32), jnp.int32)
    return jnp.bitwise_or(jnp.bitwise_and(bi, jnp.int32(-65536)),
                          jax.lax.shift_right_logical(ai, 16))


def _half_tables(y):
    tl = _pack_i32(y[:, 0:16], y[:, 16:32])
    tr = _pack_i32(y[:, 32:48], y[:, 48:64])
    return tl, tr


def _t0_body(x_ref, w_ref, b_ref, *outs):
    h = jnp.dot(x_ref[...], w_ref[...], preferred_element_type=jnp.float32)
    h = jnp.maximum(h + b_ref[...], 0.0)
    for n in range(N_NETS):
        y = h[:, n * HID:(n + 1) * HID]
        outs[2 * n][...] = y[:, :HALF]
        outs[2 * n + 1][...] = y[:, HALF:]
        tl, tr = _half_tables(y)
        outs[2 * N_NETS + 2 * n][...] = tl
        outs[2 * N_NETS + 2 * n + 1][...] = tr


def _dense0(x, w0cat, b0cat):
    return pl.pallas_call(
        _t0_body,
        grid=(RBLK,),
        in_specs=[
            pl.BlockSpec((RT, IN_C), lambda i: (i, 0)),
            pl.BlockSpec((IN_C, N_NETS * HID), lambda i: (0, 0)),
            pl.BlockSpec((1, N_NETS * HID), lambda i: (0, 0)),
        ],
        out_specs=[pl.BlockSpec((RT, HALF), lambda i: (i, 0))] * (2 * N_NETS)
        + [pl.BlockSpec((RT, HALF // 2), lambda i: (i, 0))] * (2 * N_NETS),
        out_shape=[jax.ShapeDtypeStruct((N, HALF), jnp.float32)] * (2 * N_NETS)
        + [jax.ShapeDtypeStruct((N, HALF // 2), jnp.int32)] * (2 * N_NETS),
    )(x, w0cat, b0cat)


def _t1_body(aggl, aggr, x0l, x0r, wc, out_l, out_r, out_tl, out_tr, *, beta):
    agg = jnp.concatenate([aggl[...], aggr[...]], axis=1)
    x0 = jnp.concatenate([x0l[...], x0r[...]], axis=1)
    h = (1.0 - ALPHA) * agg + ALPHA * x0
    y = (1.0 - beta) * h + beta * jnp.dot(
        h, wc[...], preferred_element_type=jnp.float32)
    y = jnp.maximum(y, 0.0)
    out_l[...] = y[:, :HALF]
    out_r[...] = y[:, HALF:]
    tl, tr = _half_tables(y)
    out_tl[...] = tl
    out_tr[...] = tr


def _dense_layer(aggl, aggr, x0l, x0r, wc, beta):
    return pl.pallas_call(
        functools.partial(_t1_body, beta=beta),
        grid=(RBLK,),
        in_specs=[
            pl.BlockSpec((RT, HALF), lambda i: (i, 0)),
            pl.BlockSpec((RT, HALF), lambda i: (i, 0)),
            pl.BlockSpec((RT, HALF), lambda i: (i, 0)),
            pl.BlockSpec((RT, HALF), lambda i: (i, 0)),
            pl.BlockSpec((HID, HID), lambda i: (0, 0)),
        ],
        out_specs=[pl.BlockSpec((RT, HALF), lambda i: (i, 0))] * 2
        + [pl.BlockSpec((RT, HALF // 2), lambda i: (i, 0))] * 2,
        out_shape=[jax.ShapeDtypeStruct((N, HALF), jnp.float32)] * 2
        + [jax.ShapeDtypeStruct((N, HALF // 2), jnp.int32)] * 2,
    )(aggl, aggr, x0l, x0r, wc)


def _t2_body(x0l, x0r, x1l, x1r, x2l, x2r, w1s, b1s, out):
    halves = [(x0l, x0r), (x1l, x1r), (x2l, x2r)]
    acc = jnp.zeros((RT, OUT_C), jnp.float32)
    for n in range(N_NETS):
        xn = jnp.concatenate([halves[n][0][...], halves[n][1][...]], axis=1)
        o = jnp.dot(xn, w1s[...][n], preferred_element_type=jnp.float32)
        o = o + b1s[...][n][None, :]
        m = jnp.max(o, axis=-1, keepdims=True)
        ls = m + jnp.log(jnp.sum(jnp.exp(o - m), axis=-1, keepdims=True))
        acc = acc + (o - ls)
    out[...] = acc * (1.0 / N_NETS)


def _dense_out(xs, w1s, b1s):
    return pl.pallas_call(
        _t2_body,
        grid=(RBLK,),
        in_specs=[pl.BlockSpec((RT, HALF), lambda i: (i, 0))] * (2 * N_NETS) + [
            pl.BlockSpec((N_NETS, HID, OUT_C), lambda i: (0, 0, 0)),
            pl.BlockSpec((N_NETS, OUT_C), lambda i: (0, 0)),
        ],
        out_specs=pl.BlockSpec((RT, OUT_C), lambda i: (i, 0)),
        out_shape=jax.ShapeDtypeStruct((N, OUT_C), jnp.float32),
    )(*xs, w1s, b1s)


def kernel(x, edge_index, params):
    src = edge_index[0].astype(jnp.int32)
    dst = edge_index[1].astype(jnp.int32)
    src_pad = jnp.concatenate(
        [src, jnp.zeros((E_PAD + SRC_EXTRA - E,), jnp.int32)]
    ).reshape(-1, 1, BLK)
    dst_pad = jnp.concatenate(
        [dst, jnp.full((E_PAD + SRC_EXTRA - E,), DUMMY, jnp.int32)]
    ).reshape(-1, 1, BLK)
    cidx = jnp.concatenate([src_pad, dst_pad], axis=1)

    w0cat = jnp.concatenate([p["W0"] for p in params], axis=1)
    b0cat = jnp.concatenate([p["b0"] for p in params]).reshape(1, -1)
    w1s = jnp.stack([p["W1"] for p in params])
    b1s = jnp.stack([p["b1"] for p in params])

    zeros_tile = jnp.zeros((ROWS_PER_TILE, HALF), jnp.float32)

    t0_out = _dense0(x, w0cat, b0cat)
    x0h = list(t0_out[:2 * N_NETS])
    xh = list(x0h)
    xbh = list(t0_out[2 * N_NETS:])
    for l in range(NUM_LAYERS):
        beta = math.log(THETA / (l + 1) + 1.0)
        for n in range(N_NETS):
            aggl, aggr = _sc_propagate(xbh[2 * n], xbh[2 * n + 1],
                                       cidx, zeros_tile)
            xl, xr, tl, tr = _dense_layer(aggl, aggr, x0h[2 * n],
                                          x0h[2 * n + 1],
                                          params[n]["Wc"][l], beta)
            xh[2 * n] = xl
            xh[2 * n + 1] = xr
            xbh[2 * n] = tl
            xbh[2 * n + 1] = tr
    return _dense_out(xh, w1s, b1s)


# speedup vs baseline: 1.6534x; 1.6534x over previous
import functools
import math

import jax
import jax.numpy as jnp
from jax import lax
from jax.experimental import pallas as pl
from jax.experimental.pallas import tpu as pltpu
from jax.experimental.pallas import tpu_sc as plsc

N = 50000
E = 800000
IN_C = 128
HID = 64
OUT_C = 64
NUM_LAYERS = 4
N_NETS = 3
ALPHA = 0.1
THETA = 0.5

HALF = HID // 2
NSUB = 16
BLK = 96
K = 3
NCHUNK = 174
NBLK = K * NCHUNK
E_PAD = NSUB * NBLK * BLK
SRC_EXTRA = 2 * K * BLK
ROWS_PER_TILE = 3126
R_PAD = NSUB * ROWS_PER_TILE
DUMMY = 50008
RBLK = 50
RT = N // RBLK

_MESH = plsc.VectorSubcoreMesh(
    core_axis_name="c", subcore_axis_name="s", num_cores=2, num_subcores=16
)


def _sc_body(xl_hbm, xr_hbm, cidx_hbm, zeros_hbm, outl_hbm, outr_hbm,
             cidx, rows_bf, rows, acc, gsem, ssem, isem):
    c = lax.axis_index("c")
    s = lax.axis_index("s")

    r0 = s * ROWS_PER_TILE
    pltpu.sync_copy(zeros_hbm, acc.at[pl.ds(r0, ROWS_PER_TILE)])

    plsc.subcore_barrier()

    def edge_loop(x_hbm):
        rbase = s * NBLK

        def load_cidx(t, slot):
            return pltpu.async_copy(
                cidx_hbm.at[pl.ds(rbase + t * K, K)], cidx.at[slot], isem)

        def drain_cidx():
            pltpu.make_async_copy(
                cidx_hbm.at[pl.ds(0, K)], cidx.at[0], isem).wait()

        def fire_gathers(bslot, islot):
            for j in range(K):
                pltpu.async_copy(x_hbm.at[cidx.at[islot, j, 0]],
                                 rows_bf.at[bslot, j], gsem)

        def drain_gathers(bslot):
            for j in range(K):
                pltpu.make_async_copy(x_hbm.at[pl.ds(0, BLK)],
                                      rows_bf.at[bslot, j], gsem).wait()

        def convert(bslot, slot):
            mask = jnp.int32(-65536)
            for j in range(K):
                @plsc.parallel_loop(0, BLK, unroll=4)
                def _cv(r):
                    w = rows_bf[bslot, j, r, pl.ds(0, 16)]
                    a = plsc.bitcast(jnp.left_shift(w, 16), jnp.float32)
                    b = plsc.bitcast(jnp.bitwise_and(w, mask), jnp.float32)
                    rows[slot, j, r, pl.ds(0, 16)] = a
                    rows[slot, j, r, pl.ds(16, 16)] = b

        def fire_scatters(slot, islot):
            for j in range(K):
                pltpu.async_copy(rows.at[slot, j], acc.at[cidx.at[islot, j, 1]],
                                 ssem, add=True)

        def drain_scatters(slot):
            for j in range(K):
                pltpu.make_async_copy(zeros_hbm.at[pl.ds(0, BLK)],
                                      rows.at[slot, j], ssem).wait()

        load_cidx(0, 0).wait()
        fire_gathers(0, 0)
        load_cidx(1, 1)

        @pl.loop(0, NCHUNK - 1)
        def _ch(t):
            q = t % 2
            q3 = t % 3

            drain_gathers(q)

            @pl.when(t != 0)
            def _ds():
                drain_scatters(1 - q)

            drain_cidx()
            fire_gathers(1 - q, (t + 1) % 3)
            load_cidx(t + 2, (t + 2) % 3)
            convert(q, q)
            fire_scatters(q, q3)

        L = NCHUNK - 1
        drain_gathers(L % 2)
        drain_scatters(1 - L % 2)
        convert(L % 2, L % 2)
        fire_scatters(L % 2, L % 3)
        drain_scatters(L % 2)
        drain_cidx()

    @pl.when(c == 0)
    def _e0():
        edge_loop(xl_hbm)

    @pl.when(c == 1)
    def _e1():
        edge_loop(xr_hbm)

    plsc.subcore_barrier()

    @pl.when(c == 0)
    def _f0():
        pltpu.sync_copy(acc.at[pl.ds(r0, ROWS_PER_TILE)],
                        outl_hbm.at[pl.ds(r0, ROWS_PER_TILE)])

    @pl.when(c == 1)
    def _f1():
        pltpu.sync_copy(acc.at[pl.ds(r0, ROWS_PER_TILE)],
                        outr_hbm.at[pl.ds(r0, ROWS_PER_TILE)])


_sc_propagate = pl.kernel(
    _sc_body,
    out_type=[jax.ShapeDtypeStruct((R_PAD, HALF), jnp.float32),
              jax.ShapeDtypeStruct((R_PAD, HALF), jnp.float32)],
    mesh=_MESH,
    scratch_types=[
        pltpu.VMEM((3, K, 2, BLK), jnp.int32),
        pltpu.VMEM((2, K, BLK, HALF // 2), jnp.int32),
        pltpu.VMEM((2, K, BLK, HALF), jnp.float32),
        pltpu.VMEM_SHARED((R_PAD, HALF), jnp.float32),
        pltpu.SemaphoreType.DMA,
        pltpu.SemaphoreType.DMA,
        pltpu.SemaphoreType.DMA,
    ],
    compiler_params=pltpu.CompilerParams(use_tc_tiling_on_sc=False,
                                         needs_layout_passes=False),
)


def _pack_i32(a, b):
    ai = jax.lax.bitcast_convert_type(
        a.astype(jnp.bfloat16).astype(jnp.float32), jnp.int32)
    bi = jax.lax.bitcast_convert_type(
        b.astype(jnp.bfloat16).astype(jnp.float32), jnp.int32)
    return jnp.bitwise_or(jnp.bitwise_and(bi, jnp.int32(-65536)),
                          jax.lax.shift_right_logical(ai, 16))


def _half_tables(y):
    tl = _pack_i32(y[:, 0:16], y[:, 16:32])
    tr = _pack_i32(y[:, 32:48], y[:, 48:64])
    return tl, tr


def _t0_body(x_ref, w_ref, b_ref, *outs):
    h = jnp.dot(x_ref[...], w_ref[...], preferred_element_type=jnp.float32)
    h = jnp.maximum(h + b_ref[...], 0.0)
    for n in range(N_NETS):
        y = h[:, n * HID:(n + 1) * HID]
        outs[2 * n][...] = y[:, :HALF]
        outs[2 * n + 1][...] = y[:, HALF:]
        tl, tr = _half_tables(y)
        outs[2 * N_NETS + 2 * n][...] = tl
        outs[2 * N_NETS + 2 * n + 1][...] = tr


def _dense0(x, w0cat, b0cat):
    return pl.pallas_call(
        _t0_body,
        grid=(RBLK,),
        in_specs=[
            pl.BlockSpec((RT, IN_C), lambda i: (i, 0)),
            pl.BlockSpec((IN_C, N_NETS * HID), lambda i: (0, 0)),
            pl.BlockSpec((1, N_NETS * HID), lambda i: (0, 0)),
        ],
        out_specs=[pl.BlockSpec((RT, HALF), lambda i: (i, 0))] * (2 * N_NETS)
        + [pl.BlockSpec((RT, HALF // 2), lambda i: (i, 0))] * (2 * N_NETS),
        out_shape=[jax.ShapeDtypeStruct((N, HALF), jnp.float32)] * (2 * N_NETS)
        + [jax.ShapeDtypeStruct((N, HALF // 2), jnp.int32)] * (2 * N_NETS),
    )(x, w0cat, b0cat)


def _t1_body(aggl, aggr, x0l, x0r, wc, out_l, out_r, out_tl, out_tr, *, beta):
    agg = jnp.concatenate([aggl[...], aggr[...]], axis=1)
    x0 = jnp.concatenate([x0l[...], x0r[...]], axis=1)
    h = (1.0 - ALPHA) * agg + ALPHA * x0
    y = (1.0 - beta) * h + beta * jnp.dot(
        h, wc[...], preferred_element_type=jnp.float32)
    y = jnp.maximum(y, 0.0)
    out_l[...] = y[:, :HALF]
    out_r[...] = y[:, HALF:]
    tl, tr = _half_tables(y)
    out_tl[...] = tl
    out_tr[...] = tr


def _dense_layer(aggl, aggr, x0l, x0r, wc, beta):
    return pl.pallas_call(
        functools.partial(_t1_body, beta=beta),
        grid=(RBLK,),
        in_specs=[
            pl.BlockSpec((RT, HALF), lambda i: (i, 0)),
            pl.BlockSpec((RT, HALF), lambda i: (i, 0)),
            pl.BlockSpec((RT, HALF), lambda i: (i, 0)),
            pl.BlockSpec((RT, HALF), lambda i: (i, 0)),
            pl.BlockSpec((HID, HID), lambda i: (0, 0)),
        ],
        out_specs=[pl.BlockSpec((RT, HALF), lambda i: (i, 0))] * 2
        + [pl.BlockSpec((RT, HALF // 2), lambda i: (i, 0))] * 2,
        out_shape=[jax.ShapeDtypeStruct((N, HALF), jnp.float32)] * 2
        + [jax.ShapeDtypeStruct((N, HALF // 2), jnp.int32)] * 2,
    )(aggl, aggr, x0l, x0r, wc)


def _t2_body(x0l, x0r, x1l, x1r, x2l, x2r, w1s, b1s, out):
    halves = [(x0l, x0r), (x1l, x1r), (x2l, x2r)]
    acc = jnp.zeros((RT, OUT_C), jnp.float32)
    for n in range(N_NETS):
        xn = jnp.concatenate([halves[n][0][...], halves[n][1][...]], axis=1)
        o = jnp.dot(xn, w1s[...][n], preferred_element_type=jnp.float32)
        o = o + b1s[...][n][None, :]
        m = jnp.max(o, axis=-1, keepdims=True)
        ls = m + jnp.log(jnp.sum(jnp.exp(o - m), axis=-1, keepdims=True))
        acc = acc + (o - ls)
    out[...] = acc * (1.0 / N_NETS)


def _dense_out(xs, w1s, b1s):
    return pl.pallas_call(
        _t2_body,
        grid=(RBLK,),
        in_specs=[pl.BlockSpec((RT, HALF), lambda i: (i, 0))] * (2 * N_NETS) + [
            pl.BlockSpec((N_NETS, HID, OUT_C), lambda i: (0, 0, 0)),
            pl.BlockSpec((N_NETS, OUT_C), lambda i: (0, 0)),
        ],
        out_specs=pl.BlockSpec((RT, OUT_C), lambda i: (i, 0)),
        out_shape=jax.ShapeDtypeStruct((N, OUT_C), jnp.float32),
    )(*xs, w1s, b1s)


def kernel(x, edge_index, params):
    src = edge_index[0].astype(jnp.int32)
    dst = edge_index[1].astype(jnp.int32)
    src_pad = jnp.concatenate(
        [src, jnp.zeros((E_PAD + SRC_EXTRA - E,), jnp.int32)]
    ).reshape(-1, 1, BLK)
    dst_pad = jnp.concatenate(
        [dst, jnp.full((E_PAD + SRC_EXTRA - E,), DUMMY, jnp.int32)]
    ).reshape(-1, 1, BLK)
    cidx = jnp.concatenate([src_pad, dst_pad], axis=1)

    w0cat = jnp.concatenate([p["W0"] for p in params], axis=1)
    b0cat = jnp.concatenate([p["b0"] for p in params]).reshape(1, -1)
    w1s = jnp.stack([p["W1"] for p in params])
    b1s = jnp.stack([p["b1"] for p in params])

    zeros_tile = jnp.zeros((ROWS_PER_TILE, HALF), jnp.float32)

    t0_out = _dense0(x, w0cat, b0cat)
    x0h = list(t0_out[:2 * N_NETS])
    xh = list(x0h)
    xbh = list(t0_out[2 * N_NETS:])
    for l in range(NUM_LAYERS):
        beta = math.log(THETA / (l + 1) + 1.0)
        for n in range(N_NETS):
            aggl, aggr = _sc_propagate(xbh[2 * n], xbh[2 * n + 1],
                                       cidx, zeros_tile)
            xl, xr, tl, tr = _dense_layer(aggl, aggr, x0h[2 * n],
                                          x0h[2 * n + 1],
                                          params[n]["Wc"][l], beta)
            xh[2 * n] = xl
            xh[2 * n + 1] = xr
            xbh[2 * n] = tl
            xbh[2 * n + 1] = tr
    return _dense_out(xh, w1s, b1s)
